# baseline (speedup 1.0000x reference)
import functools

import jax
import jax.numpy as jnp
from jax import lax
from jax.experimental import pallas as pl
from jax.experimental.pallas import tpu as pltpu
from jax.experimental.pallas import tpu_sc as plsc

_NCHUNK = 4


def kernel(x, table):
    B = x.shape[0]
    V, D = table.shape

    info = plsc.get_sparse_core_info()
    NC, NS = info.num_cores, info.num_subcores
    NW = NC * NS
    assert B % (NW * _NCHUNK) == 0
    b_per_w = B // NW
    c_rows = b_per_w // _NCHUNK

    mesh = plsc.VectorSubcoreMesh(core_axis_name="c", subcore_axis_name="s")

    @functools.partial(
        pl.kernel,
        mesh=mesh,
        out_type=jax.ShapeDtypeStruct((B, D), jnp.float32),
        scratch_types=[
            pltpu.VMEM((b_per_w,), jnp.int32),
            [pltpu.VMEM((c_rows, D), jnp.float32) for _ in range(_NCHUNK)],
            pltpu.SemaphoreType.DMA,
            pltpu.SemaphoreType.DMA,
        ],
        compiler_params=pltpu.CompilerParams(use_tc_tiling_on_sc=False),
    )
    def gather_kernel(table_hbm, idx_hbm, out_hbm, idx_v, bufs, sem_g, sem_s):
        wid = lax.axis_index("s") * NC + lax.axis_index("c")
        base = wid * b_per_w
        pltpu.sync_copy(idx_hbm.at[pl.ds(base, b_per_w)], idx_v)
        gathers = []
        for k in range(_NCHUNK):
            gathers.append(
                pltpu.async_copy(
                    table_hbm.at[idx_v.at[pl.ds(k * c_rows, c_rows)]],
                    bufs[k],
                    sem_g,
                )
            )
        stores = []
        for k in range(_NCHUNK):
            gathers[k].wait()
            stores.append(
                pltpu.async_copy(
                    bufs[k],
                    out_hbm.at[pl.ds(base + k * c_rows, c_rows)],
                    sem_s,
                )
            )
        for k in range(_NCHUNK):
            stores[k].wait()

    return gather_kernel(table, x.astype(jnp.int32))


# speedup vs baseline: 1.0122x; 1.0122x over previous
import functools

import jax
import jax.numpy as jnp
from jax import lax
from jax.experimental import pallas as pl
from jax.experimental.pallas import tpu as pltpu
from jax.experimental.pallas import tpu_sc as plsc

_NCHUNK = 2


def kernel(x, table):
    B = x.shape[0]
    V, D = table.shape

    info = plsc.get_sparse_core_info()
    NC, NS = info.num_cores, info.num_subcores
    NW = NC * NS
    assert B % (NW * _NCHUNK) == 0
    b_per_w = B // NW
    c_rows = b_per_w // _NCHUNK

    mesh = plsc.VectorSubcoreMesh(core_axis_name="c", subcore_axis_name="s")

    @functools.partial(
        pl.kernel,
        mesh=mesh,
        out_type=jax.ShapeDtypeStruct((B, D), jnp.float32),
        scratch_types=[
            pltpu.VMEM((b_per_w,), jnp.int32),
            [pltpu.VMEM((c_rows, D), jnp.float32) for _ in range(_NCHUNK)],
            pltpu.SemaphoreType.DMA,
            pltpu.SemaphoreType.DMA,
        ],
        compiler_params=pltpu.CompilerParams(use_tc_tiling_on_sc=False),
    )
    def gather_kernel(table_hbm, idx_hbm, out_hbm, idx_v, bufs, sem_g, sem_s):
        wid = lax.axis_index("s") * NC + lax.axis_index("c")
        base = wid * b_per_w
        pltpu.sync_copy(idx_hbm.at[pl.ds(base, b_per_w)], idx_v)
        gathers = []
        for k in range(_NCHUNK):
            gathers.append(
                pltpu.async_copy(
                    table_hbm.at[idx_v.at[pl.ds(k * c_rows, c_rows)]],
                    bufs[k],
                    sem_g,
                )
            )
        stores = []
        for k in range(_NCHUNK):
            gathers[k].wait()
            stores.append(
                pltpu.async_copy(
                    bufs[k],
                    out_hbm.at[pl.ds(base + k * c_rows, c_rows)],
                    sem_s,
                )
            )
        for k in range(_NCHUNK):
            stores[k].wait()

    return gather_kernel(table, x.astype(jnp.int32))


# speedup vs baseline: 1.0271x; 1.0147x over previous
import functools

import jax
import jax.numpy as jnp
from jax import lax
from jax.experimental import pallas as pl
from jax.experimental.pallas import tpu as pltpu
from jax.experimental.pallas import tpu_sc as plsc


def kernel(x, table):
    B = x.shape[0]
    V, D = table.shape

    info = plsc.get_sparse_core_info()
    NC, NS = info.num_cores, info.num_subcores
    NW = NC * NS
    assert B % NW == 0
    b_per_w = B // NW

    mesh = plsc.VectorSubcoreMesh(core_axis_name="c", subcore_axis_name="s")

    @functools.partial(
        pl.kernel,
        mesh=mesh,
        out_type=jax.ShapeDtypeStruct((B, D), jnp.float32),
        scratch_types=[
            pltpu.VMEM((b_per_w,), jnp.int32),
            pltpu.VMEM((b_per_w, D), jnp.float32),
            pltpu.SemaphoreType.DMA,
        ],
        compiler_params=pltpu.CompilerParams(
            use_tc_tiling_on_sc=False,
            disable_bounds_checks=True,
            disable_semaphore_checks=True,
        ),
    )
    def gather_kernel(table_hbm, idx_hbm, out_hbm, idx_v, rows_v, sem):
        wid = lax.axis_index("s") * NC + lax.axis_index("c")
        base = wid * b_per_w
        pltpu.sync_copy(idx_hbm.at[pl.ds(base, b_per_w)], idx_v)
        pltpu.async_copy(table_hbm.at[idx_v], rows_v, sem).wait()
        pltpu.sync_copy(rows_v, out_hbm.at[pl.ds(base, b_per_w)])

    return gather_kernel(table, x.astype(jnp.int32))


# speedup vs baseline: 1.0469x; 1.0193x over previous
import functools

import jax
import jax.numpy as jnp
from jax import lax
from jax.experimental import pallas as pl
from jax.experimental.pallas import tpu as pltpu
from jax.experimental.pallas import tpu_sc as plsc


def kernel(x, table):
    B = x.shape[0]
    V, D = table.shape

    info = plsc.get_sparse_core_info()
    NS = info.num_subcores
    NW = NS
    assert B % NW == 0
    b_per_w = B // NW

    mesh = plsc.VectorSubcoreMesh(
        core_axis_name="c", subcore_axis_name="s", num_cores=1
    )

    @functools.partial(
        pl.kernel,
        mesh=mesh,
        out_type=jax.ShapeDtypeStruct((B, D), jnp.float32),
        scratch_types=[
            pltpu.VMEM((b_per_w,), jnp.int32),
            pltpu.VMEM((b_per_w, D), jnp.float32),
            pltpu.SemaphoreType.DMA,
            pltpu.SemaphoreType.DMA,
        ],
        compiler_params=pltpu.CompilerParams(use_tc_tiling_on_sc=False),
    )
    def gather_kernel(table_hbm, idx_hbm, out_hbm, idx_v, rows_v, sem_g, sem_s):
        wid = lax.axis_index("s")
        base = wid * b_per_w
        half = b_per_w // 2
        pltpu.sync_copy(idx_hbm.at[pl.ds(base, b_per_w)], idx_v)
        g0 = pltpu.async_copy(
            table_hbm.at[idx_v.at[pl.ds(0, half)]],
            rows_v.at[pl.ds(0, half)], sem_g)
        g1 = pltpu.async_copy(
            table_hbm.at[idx_v.at[pl.ds(half, half)]],
            rows_v.at[pl.ds(half, half)], sem_g)
        g0.wait()
        s0 = pltpu.async_copy(
            rows_v.at[pl.ds(0, half)],
            out_hbm.at[pl.ds(base, half)], sem_s)
        g1.wait()
        s1 = pltpu.async_copy(
            rows_v.at[pl.ds(half, half)],
            out_hbm.at[pl.ds(base + half, half)], sem_s)
        s0.wait()
        s1.wait()

    return gather_kernel(table, x.astype(jnp.int32))


# speedup vs baseline: 1.0544x; 1.0072x over previous
import functools

import jax
import jax.numpy as jnp
from jax import lax
from jax.experimental import pallas as pl
from jax.experimental.pallas import tpu as pltpu
from jax.experimental.pallas import tpu_sc as plsc


def kernel(x, table):
    B = x.shape[0]
    V, D = table.shape

    info = plsc.get_sparse_core_info()
    NS = info.num_subcores
    NW = NS
    assert B % NW == 0
    b_per_w = B // NW

    mesh = plsc.VectorSubcoreMesh(
        core_axis_name="c", subcore_axis_name="s", num_cores=1
    )

    @functools.partial(
        pl.kernel,
        mesh=mesh,
        out_type=jax.ShapeDtypeStruct((B, D), jnp.float32),
        scratch_types=[
            pltpu.VMEM((b_per_w,), jnp.int32),
            pltpu.VMEM((b_per_w, D), jnp.float32),
            pltpu.SemaphoreType.DMA,
        ],
        compiler_params=pltpu.CompilerParams(
            use_tc_tiling_on_sc=False,
            skip_device_barrier=True,
        ),
    )
    def gather_kernel(table_hbm, idx_hbm, out_hbm, idx_v, rows_v, sem):
        wid = lax.axis_index("s")
        base = wid * b_per_w
        pltpu.sync_copy(idx_hbm.at[pl.ds(base, b_per_w)], idx_v)
        pltpu.async_copy(table_hbm.at[idx_v], rows_v, sem).wait()
        pltpu.sync_copy(rows_v, out_hbm.at[pl.ds(base, b_per_w)])

    return gather_kernel(table, x.astype(jnp.int32))
